# baseline (speedup 1.0000x reference)
import functools

import jax
import jax.numpy as jnp
from jax import lax
from jax.experimental import pallas as pl
from jax.experimental.pallas import tpu as pltpu
from jax.experimental.pallas import tpu_sc as plsc

_RECOMBINE_IDX = (0, 1, 2, 10, 11, 12, 0, 1, 3, 10, 11, 13,
                  0, 1, 4, 10, 11, 14, 0, 1, 5, 10, 11, 15,
                  0, 1, 6, 10, 11, 16, 0, 1, 7, 10, 11, 17,
                  0, 1, 8, 10, 11, 18, 0, 1, 9, 10, 11, 19)

_B, _S, _M, _D = 1024, 20, 20, 64
_K = len(_RECOMBINE_IDX)
_NG = _B * _S
_ROWS_OUT = _NG * _K
_NC, _NS = 2, 16
_NW = _NC * _NS
_ROWS_W = _ROWS_OUT // _NW
_C = 128
_NCH = _ROWS_W // _C
_NB = 8
_NSTEP = _NCH // _NB


def _gather_rows(x_hbm, idx_hbm, out_hbm, idx_v, bufs, gsem, ssem):
    wid = lax.axis_index("s") * _NC + lax.axis_index("c")
    base = wid * _ROWS_W
    pltpu.sync_copy(idx_hbm.at[wid], idx_v)

    def step(i, _):
        g0 = i * _NB
        gathers = []
        for b in range(_NB):
            gathers.append(pltpu.async_copy(
                x_hbm.at[idx_v.at[g0 + b]], bufs[b], gsem))
        scatters = []
        for b in range(_NB):
            gathers[b].wait()
            scatters.append(pltpu.async_copy(
                bufs[b], out_hbm.at[pl.ds(base + (g0 + b) * _C, _C)], ssem))
        for b in range(_NB):
            scatters[b].wait()
        return 0

    lax.fori_loop(0, _NSTEP, step, 0)


@functools.partial(jax.jit, static_argnums=())
def _recombine(x_flat, idx):
    mesh = plsc.VectorSubcoreMesh(
        core_axis_name="c", subcore_axis_name="s",
        num_cores=_NC, num_subcores=_NS)
    scratch = [pltpu.VMEM((_NCH, _C), jnp.int32)]
    scratch += [pltpu.VMEM((_C, _D), jnp.float32) for _ in range(_NB)]
    scratch += [pltpu.SemaphoreType.DMA, pltpu.SemaphoreType.DMA]

    def body(x_hbm, idx_hbm, out_hbm, idx_v, *rest):
        bufs, (gsem, ssem) = rest[:_NB], rest[_NB:]
        _gather_rows(x_hbm, idx_hbm, out_hbm, idx_v, bufs, gsem, ssem)

    return pl.kernel(
        body,
        out_type=jax.ShapeDtypeStruct((_ROWS_OUT, _D), jnp.float32),
        mesh=mesh,
        scratch_types=scratch,
        compiler_params=pltpu.CompilerParams(use_tc_tiling_on_sc=False),
    )(x_flat, idx)


def _build_index():
    r = jnp.arange(_ROWS_OUT, dtype=jnp.int32)
    pattern = jnp.array(_RECOMBINE_IDX, dtype=jnp.int32)
    gidx = (r // _K) * _M + pattern[r % _K]
    return gidx.reshape(_NW, _NCH, _C)


def kernel(x):
    b, s, m, d = x.shape
    x_flat = x.reshape(b * s * m, d)
    idx = _build_index()
    out = _recombine(x_flat, idx)
    return out.reshape(b, s, 8, 6, d)


# speedup vs baseline: 17.2487x; 17.2487x over previous
import functools

import jax
import jax.numpy as jnp
from jax import lax
from jax.experimental import pallas as pl
from jax.experimental.pallas import tpu as pltpu
from jax.experimental.pallas import tpu_sc as plsc

_B, _S, _M, _D = 1024, 20, 20, 64
_NP_IN = _S * _M
_NP_OUT = _S * 48
_NC, _NS = 2, 16
_NW = _NC * _NS
_NTASK = _S * 6
_TPW = (_NTASK + _NW - 1) // _NW
_H = 32


def _column_task(xt_hbm, out_hbm, bufs, sems, t):
    s_idx = t // 6
    j = t % 6
    is_const = jnp.logical_and(j != 2, j != 5)

    @pl.when(is_const)
    def _():
        base = jnp.where(j == 0, 0, jnp.where(j == 1, 1,
                         jnp.where(j == 3, 10, 11)))
        src = s_idx * _M + base
        writes = []
        for h in range(2):
            pltpu.sync_copy(xt_hbm.at[src, pl.ds(h * _H, _H)], bufs[h])
            for k in range(8):
                dst = s_idx * 48 + k * 6 + j
                writes.append(pltpu.async_copy(
                    bufs[h], out_hbm.at[dst, pl.ds(h * _H, _H)], sems[h]))
        for wdma in writes:
            wdma.wait()

    @pl.when(jnp.logical_not(is_const))
    def _():
        base = jnp.where(j == 2, 2, 12)
        writes = [None, None, None]
        for k in range(8):
            src = s_idx * _M + base + k
            dst = s_idx * 48 + k * 6 + j
            for h in range(2):
                c = 2 * k + h
                r = c % 3
                if writes[r] is not None:
                    writes[r].wait()
                pltpu.sync_copy(xt_hbm.at[src, pl.ds(h * _H, _H)], bufs[r])
                writes[r] = pltpu.async_copy(
                    bufs[r], out_hbm.at[dst, pl.ds(h * _H, _H)], sems[r])
        for wdma in writes:
            wdma.wait()


def _body(xt_hbm, out_hbm, b0, b1, b2, s0, s1, s2):
    wid = lax.axis_index("s") * _NC + lax.axis_index("c")
    bufs, sems = (b0, b1, b2), (s0, s1, s2)
    for i in range(_TPW):
        t = wid + _NW * i

        @pl.when(t < _NTASK)
        def _():
            _column_task(xt_hbm, out_hbm, bufs, sems, t)


@jax.jit
def _recombine(xt):
    mesh = plsc.VectorSubcoreMesh(
        core_axis_name="c", subcore_axis_name="s",
        num_cores=_NC, num_subcores=_NS)
    scratch = [pltpu.VMEM((_H, _B), jnp.float32) for _ in range(3)]
    scratch += [pltpu.SemaphoreType.DMA] * 3
    return pl.kernel(
        _body,
        out_type=jax.ShapeDtypeStruct((_NP_OUT, _D, _B), jnp.float32),
        mesh=mesh,
        scratch_types=scratch,
    )(xt)


def kernel(x):
    b, s, m, d = x.shape
    xt = jnp.transpose(x, (1, 2, 3, 0)).reshape(s * m, d, b)
    out = _recombine(xt)
    out = out.reshape(s, 8, 6, d, b).transpose(4, 0, 1, 2, 3)
    return out


# speedup vs baseline: 19.5660x; 1.1343x over previous
import functools

import jax
import jax.numpy as jnp
from jax import lax
from jax.experimental import pallas as pl
from jax.experimental.pallas import tpu as pltpu
from jax.experimental.pallas import tpu_sc as plsc

_B, _S, _M, _D = 1024, 20, 20, 64
_NP_IN = _S * _M
_NP_OUT = _S * 48
_NC, _NS = 2, 16
_NW = _NC * _NS
_NTASK = _S * 6
_TPW = (_NTASK + _NW - 1) // _NW
_H = 32


def _bcast_task(xt_hbm, out_hbm, bufs, sems, u):
    s_idx = u // 4
    j4 = u % 4
    base = jnp.where(j4 == 0, 0, jnp.where(j4 == 1, 1,
                     jnp.where(j4 == 2, 10, 11)))
    j = jnp.where(j4 == 0, 0, jnp.where(j4 == 1, 1,
                  jnp.where(j4 == 2, 3, 4)))
    src = s_idx * _M + base
    writes = []
    for h in range(2):
        pltpu.sync_copy(xt_hbm.at[src, pl.ds(h * _H, _H)], bufs[h])
        for k in range(8):
            dst = s_idx * 48 + k * 6 + j
            writes.append(pltpu.async_copy(
                bufs[h], out_hbm.at[dst, pl.ds(h * _H, _H)], sems[h]))
    for wdma in writes:
        wdma.wait()


def _uniq_task(xt_hbm, out_hbm, bufs, sems, v):
    s_idx = v // 4
    q = v % 4
    j = jnp.where(q < 2, 2, 5)
    base = jnp.where(q < 2, 2, 12)
    k0 = jnp.where(q % 2 == 0, 0, 4)
    writes = [None, None, None]
    for kk in range(4):
        k = k0 + kk
        src = s_idx * _M + base + k
        dst = s_idx * 48 + k * 6 + j
        for h in range(2):
            c = 2 * kk + h
            r = c % 3
            if writes[r] is not None:
                writes[r].wait()
            pltpu.sync_copy(xt_hbm.at[src, pl.ds(h * _H, _H)], bufs[r])
            writes[r] = pltpu.async_copy(
                bufs[r], out_hbm.at[dst, pl.ds(h * _H, _H)], sems[r])
    for wdma in writes:
        if wdma is not None:
            wdma.wait()


def _body(xt_hbm, out_hbm, b0, b1, b2, s0, s1, s2):
    wid = lax.axis_index("s") * _NC + lax.axis_index("c")
    bufs, sems = (b0, b1, b2), (s0, s1, s2)
    for i in range(5):
        u = wid + _NW * i

        @pl.when(u < 80)
        def _():
            _bcast_task(xt_hbm, out_hbm, bufs, sems, u)

        @pl.when(u >= 80)
        def _():
            _uniq_task(xt_hbm, out_hbm, bufs, sems, u - 80)


@jax.jit
def _recombine(xt):
    mesh = plsc.VectorSubcoreMesh(
        core_axis_name="c", subcore_axis_name="s",
        num_cores=_NC, num_subcores=_NS)
    scratch = [pltpu.VMEM((_H, _B), jnp.float32) for _ in range(3)]
    scratch += [pltpu.SemaphoreType.DMA] * 3
    return pl.kernel(
        _body,
        out_type=jax.ShapeDtypeStruct((_NP_OUT, _D, _B), jnp.float32),
        mesh=mesh,
        scratch_types=scratch,
    )(xt)


def kernel(x):
    b, s, m, d = x.shape
    xt = jnp.transpose(x, (1, 2, 3, 0)).reshape(s * m, d, b)
    out = _recombine(xt)
    out = out.reshape(s, 8, 6, d, b).transpose(4, 0, 1, 2, 3)
    return out
